# Initial kernel scaffold; baseline (speedup 1.0000x reference)
import jax
import jax.numpy as jnp
from jax.experimental import pallas as pl


def kernel(query_ids, query_embed, key_ids, key_embed):
    raise NotImplementedError("write your pallas kernel here")


# speedup vs baseline: 33.3804x; 33.3804x over previous
import jax
import jax.numpy as jnp
from jax.experimental import pallas as pl
from jax.experimental.pallas import tpu as pltpu

N = 16384
D = 32
K = 5
TOPK = K + 1
BQ = 256
G = N // BQ


def _normalize_kernel(x_ref, o_ref):
    x = x_ref[...]
    n = jnp.sqrt(jnp.sum(x * x, axis=1, keepdims=True))
    o_ref[...] = x / jnp.maximum(n, 1e-12)


def _knn_kernel(qid_ref, qn_ref, kid_ref, knt_ref, out_ref):
    qn = qn_ref[...]
    knt = knt_ref[...]
    sims = jax.lax.dot_general(
        qn, knt, (((1,), (0,)), ((), ())),
        preferred_element_type=jnp.float32)

    qid = qid_ref[...]
    kid = kid_ref[...]
    match = (qid == kid)

    iota = jax.lax.broadcasted_iota(jnp.int32, (BQ, N), 1)
    acc = jnp.zeros((BQ, 1), jnp.float32)
    for k in range(TOPK):
        m = jnp.max(sims, axis=1, keepdims=True)
        idx = jnp.min(jnp.where(sims == m, iota, N), axis=1,
                      keepdims=True)
        chosen = iota == idx
        if k > 0:
            acc = acc + jnp.sum(
                jnp.where(chosen & match, 1.0, 0.0), axis=1, keepdims=True)
        if k < TOPK - 1:
            sims = jnp.where(chosen, -jnp.inf, sims)
    out_ref[...] = acc


def kernel(query_ids, query_embed, key_ids, key_embed):
    norm = pl.pallas_call(
        _normalize_kernel,
        grid=(G,),
        in_specs=[pl.BlockSpec((BQ, D), lambda i: (i, 0))],
        out_specs=pl.BlockSpec((BQ, D), lambda i: (i, 0)),
        out_shape=jax.ShapeDtypeStruct((N, D), jnp.float32),
        compiler_params=pltpu.CompilerParams(
            dimension_semantics=("parallel",)),
    )
    qn = norm(query_embed)
    kn = norm(key_embed)
    knt = kn.T

    counts = pl.pallas_call(
        _knn_kernel,
        grid=(G,),
        in_specs=[
            pl.BlockSpec((BQ, 1), lambda i: (i, 0)),
            pl.BlockSpec((BQ, D), lambda i: (i, 0)),
            pl.BlockSpec((1, N), lambda i: (0, 0)),
            pl.BlockSpec((D, N), lambda i: (0, 0)),
        ],
        out_specs=pl.BlockSpec((BQ, 1), lambda i: (i, 0)),
        out_shape=jax.ShapeDtypeStruct((N, 1), jnp.float32),
        compiler_params=pltpu.CompilerParams(
            dimension_semantics=("parallel",)),
    )(query_ids.reshape(N, 1), qn, key_ids.reshape(1, N), knt)

    return jnp.sum(counts) / jnp.float32(N * K)


# speedup vs baseline: 57.1154x; 1.7110x over previous
import jax
import jax.numpy as jnp
from jax.experimental import pallas as pl
from jax.experimental.pallas import tpu as pltpu

N = 16384
D = 32
K = 5
TOPK = K + 1
BQ = 256
G = N // BQ


def _normalize_kernel(x_ref, o_ref):
    x = x_ref[...]
    n = jnp.sqrt(jnp.sum(x * x, axis=1, keepdims=True))
    o_ref[...] = x / jnp.maximum(n, 1e-12)


def _knn_kernel(qid_ref, qn_ref, kid_ref, knt_ref, out_ref):
    qn = qn_ref[...]
    knt = knt_ref[...]
    sims = jax.lax.dot_general(
        qn, knt, (((1,), (0,)), ((), ())),
        preferred_element_type=jnp.float32)

    qid = qid_ref[...]
    kid = kid_ref[...]
    match = (qid == kid)

    iota2 = jax.lax.broadcasted_iota(jnp.int32, (BQ, N), 1) * 2 + 1
    piota = jnp.where(match, iota2 - 1, iota2)

    acc = jnp.zeros((BQ, 1), jnp.int32)
    for k in range(TOPK):
        m = jnp.max(sims, axis=1, keepdims=True)
        key = jnp.min(jnp.where(sims == m, piota, 2 * N), axis=1,
                      keepdims=True)
        if k > 0:
            acc = acc + (1 - (key & 1))
        if k < TOPK - 1:
            sims = jnp.where(piota == key, -jnp.inf, sims)
    out_ref[...] = acc.astype(jnp.float32)


def kernel(query_ids, query_embed, key_ids, key_embed):
    norm = pl.pallas_call(
        _normalize_kernel,
        grid=(G,),
        in_specs=[pl.BlockSpec((BQ, D), lambda i: (i, 0))],
        out_specs=pl.BlockSpec((BQ, D), lambda i: (i, 0)),
        out_shape=jax.ShapeDtypeStruct((N, D), jnp.float32),
        compiler_params=pltpu.CompilerParams(
            dimension_semantics=("parallel",)),
    )
    qn = norm(query_embed)
    kn = norm(key_embed)
    knt = kn.T

    counts = pl.pallas_call(
        _knn_kernel,
        grid=(G,),
        in_specs=[
            pl.BlockSpec((BQ, 1), lambda i: (i, 0)),
            pl.BlockSpec((BQ, D), lambda i: (i, 0)),
            pl.BlockSpec((1, N), lambda i: (0, 0)),
            pl.BlockSpec((D, N), lambda i: (0, 0)),
        ],
        out_specs=pl.BlockSpec((BQ, 1), lambda i: (i, 0)),
        out_shape=jax.ShapeDtypeStruct((N, 1), jnp.float32),
        compiler_params=pltpu.CompilerParams(
            dimension_semantics=("parallel",)),
    )(query_ids.reshape(N, 1), qn, key_ids.reshape(1, N), knt)

    return jnp.sum(counts) / jnp.float32(N * K)
